# baseline (speedup 1.0000x reference)
import functools

import jax
import jax.numpy as jnp
import numpy as np
from jax import lax
from jax.experimental import pallas as pl
from jax.experimental.pallas import tpu as pltpu
from jax.experimental.pallas import tpu_sc as plsc

E = 20000
K = 8
D = 512
DP = D // 2

NC = 2
NS = 16
NW = NC * NS
G = 8
GK = G * K
NGT = E // G
NT = 80
LANES = 16

_COLS_A = np.arange(D).reshape(D // 32, 32)[:, :16].reshape(-1)
_COLS_B = np.arange(D).reshape(D // 32, 32)[:, 16:].reshape(-1)


def _sc_gather_sum(x, nbr_flat):
    mesh = plsc.VectorSubcoreMesh(core_axis_name="c", subcore_axis_name="s")

    @functools.partial(
        pl.kernel,
        out_type=jax.ShapeDtypeStruct((E, DP), jnp.int32),
        mesh=mesh,
        scratch_types=[
            pltpu.VMEM((2, GK), jnp.int32),
            pltpu.VMEM((2, GK, D), jnp.float32),
            pltpu.VMEM((2, G, DP), jnp.int32),
            pltpu.SemaphoreType.DMA,
            pltpu.SemaphoreType.DMA,
            pltpu.SemaphoreType.DMA,
            pltpu.SemaphoreType.DMA,
            pltpu.SemaphoreType.DMA,
            pltpu.SemaphoreType.DMA,
        ],
    )
    def sc_fn(x_hbm, idx_hbm, out_hbm, idx_v, rows_v, out_v,
              si0, si1, sr0, sr1, so0, so1):
        wid = lax.axis_index("s") * NC + lax.axis_index("c")
        s_idx = (si0, si1)
        s_rows = (sr0, sr1)
        s_out = (so0, so1)

        def gof(n):
            return jnp.minimum(wid + n * NW, NGT - 1)

        def idx_dma(n, p):
            return pltpu.make_async_copy(
                idx_hbm.at[pl.ds(gof(n) * GK, GK)], idx_v.at[p], s_idx[p])

        def rows_dma(p):
            return pltpu.make_async_copy(
                x_hbm.at[idx_v.at[p]], rows_v.at[p], s_rows[p])

        def out_dma(n, p):
            return pltpu.make_async_copy(
                out_v.at[p], out_hbm.at[pl.ds(gof(n) * G, G)], s_out[p])

        def tree_sum(vals):
            while len(vals) > 1:
                nxt = [vals[k] + vals[k + 1]
                       for k in range(0, len(vals) - 1, 2)]
                if len(vals) % 2:
                    nxt.append(vals[-1])
                vals = nxt
            return vals[0]

        HMASK = jnp.int32(-65536)
        RND = jnp.int32(0x8000)

        def compute(p):
            def pos_body(q, c):
                da = pl.ds(pl.multiple_of(q * 2 * LANES, LANES), LANES)
                db = pl.ds(pl.multiple_of(q * 2 * LANES + LANES, LANES),
                           LANES)
                dw = pl.ds(pl.multiple_of(q * LANES, LANES), LANES)
                for i in range(G):
                    sa = tree_sum(
                        [rows_v[p, i * K + j, da] for j in range(K)])
                    sb = tree_sum(
                        [rows_v[p, i * K + j, db] for j in range(K)])
                    ai = lax.bitcast_convert_type(sa, jnp.int32)
                    bi = lax.bitcast_convert_type(sb, jnp.int32)
                    a16 = lax.shift_right_logical(ai + RND, 16)
                    b16 = (bi + RND) & HMASK
                    out_v[p, i, dw] = b16 | a16
                return c
            lax.fori_loop(0, DP // LANES, pos_body, 0)

        idx_dma(0, 0).start()
        idx_dma(1, 1).start()
        idx_dma(0, 0).wait()
        rows_dma(0).start()

        def pair(m, carry):
            for p in (0, 1):
                n = 2 * m + p
                if p == 0:
                    idx_dma(n + 1, 1).wait()
                    rows_dma(1).start()
                else:
                    @pl.when(m < NT // 2 - 1)
                    def _():
                        idx_dma(n + 1, 0).wait()
                        rows_dma(0).start()
                rows_dma(p).wait()
                @pl.when(m < NT // 2 - 1)
                def _():
                    idx_dma(n + 2, p).start()
                @pl.when(m >= 1)
                def _():
                    out_dma(n - 2, p).wait()
                compute(p)
                out_dma(n, p).start()
            return carry

        lax.fori_loop(0, NT // 2, pair, 0)
        out_dma(NT - 2, 0).wait()
        out_dma(NT - 1, 1).wait()

    return sc_fn(x, nbr_flat)


def _mm_body(x_ref, n_ref, w_ref, wa_ref, wb_ref, b_ref, o_ref):
    n32 = n_ref[...]
    nlo = lax.bitcast_convert_type(n32 << 16, jnp.float32)
    nhi = lax.bitcast_convert_type(n32 & jnp.int32(-65536), jnp.float32)
    dn = (((1,), (1,)), ((), ()))
    acc = lax.dot_general(x_ref[...], w_ref[...], dn,
                          preferred_element_type=jnp.float32)
    acc += lax.dot_general(nlo, wa_ref[...], dn,
                           preferred_element_type=jnp.float32)
    acc += lax.dot_general(nhi, wb_ref[...], dn,
                           preferred_element_type=jnp.float32)
    o_ref[...] = acc + (K + 1.0) * b_ref[...]


def _tc_matmul(x, n_packed, w, wa, wb, b):
    BM = 2000
    return pl.pallas_call(
        _mm_body,
        grid=(E // BM,),
        in_specs=[
            pl.BlockSpec((BM, D), lambda i: (i, 0)),
            pl.BlockSpec((BM, DP), lambda i: (i, 0)),
            pl.BlockSpec((D, D), lambda i: (0, 0)),
            pl.BlockSpec((D, DP), lambda i: (0, 0)),
            pl.BlockSpec((D, DP), lambda i: (0, 0)),
            pl.BlockSpec((1, D), lambda i: (0, 0)),
        ],
        out_specs=pl.BlockSpec((BM, D), lambda i: (i, 0)),
        out_shape=jax.ShapeDtypeStruct((E, D), jnp.float32),
    )(x, n_packed, w, wa, wb, b.reshape(1, D))


def kernel(edge_feats, neighbors, W, b):
    nbr_flat = neighbors.astype(jnp.int32).reshape(E * K)
    n_packed = _sc_gather_sum(edge_feats, nbr_flat)
    wa = W[:, _COLS_A]
    wb = W[:, _COLS_B]
    return _tc_matmul(edge_feats, n_packed, W, wa, wb, b)


# speedup vs baseline: 1.1264x; 1.1264x over previous
import functools

import jax
import jax.numpy as jnp
from jax import lax
from jax.experimental import pallas as pl
from jax.experimental.pallas import tpu as pltpu
from jax.experimental.pallas import tpu_sc as plsc

E = 20000
K = 8
D = 512

NC = 2
NS = 16
NW = NC * NS
G = 8
GK = G * K
NGT = E // G
NT = 80
LANES = 16


def _sc_gather_sum(x, nbr_flat):
    mesh = plsc.VectorSubcoreMesh(core_axis_name="c", subcore_axis_name="s")

    @functools.partial(
        pl.kernel,
        out_type=jax.ShapeDtypeStruct((E, D), jnp.float32),
        mesh=mesh,
        scratch_types=[
            pltpu.VMEM((2, GK), jnp.int32),
            pltpu.VMEM((2, GK, D), jnp.float32),
            pltpu.VMEM((2, G, D), jnp.float32),
            pltpu.SemaphoreType.DMA,
            pltpu.SemaphoreType.DMA,
            pltpu.SemaphoreType.DMA,
            pltpu.SemaphoreType.DMA,
            pltpu.SemaphoreType.DMA,
            pltpu.SemaphoreType.DMA,
        ],
    )
    def sc_fn(x_hbm, idx_hbm, out_hbm, idx_v, rows_v, out_v,
              si0, si1, sr0, sr1, so0, so1):
        wid = lax.axis_index("s") * NC + lax.axis_index("c")
        s_idx = (si0, si1)
        s_rows = (sr0, sr1)
        s_out = (so0, so1)

        def gof(n):
            return jnp.minimum(wid + n * NW, NGT - 1)

        def idx_dma(n, p):
            return pltpu.make_async_copy(
                idx_hbm.at[pl.ds(gof(n) * GK, GK)], idx_v.at[p], s_idx[p])

        def rows_dma(p):
            return pltpu.make_async_copy(
                x_hbm.at[idx_v.at[p]], rows_v.at[p], s_rows[p])

        def out_dma(n, p):
            return pltpu.make_async_copy(
                out_v.at[p], out_hbm.at[pl.ds(gof(n) * G, G)], s_out[p])

        def tree_sum(vals):
            while len(vals) > 1:
                nxt = [vals[k] + vals[k + 1]
                       for k in range(0, len(vals) - 1, 2)]
                if len(vals) % 2:
                    nxt.append(vals[-1])
                vals = nxt
            return vals[0]

        def compute(p):
            def pos_body(q, c):
                d = pl.ds(pl.multiple_of(q * LANES, LANES), LANES)
                for i in range(G):
                    out_v[p, i, d] = tree_sum(
                        [rows_v[p, i * K + j, d] for j in range(K)])
                return c
            lax.fori_loop(0, D // LANES, pos_body, 0)

        idx_dma(0, 0).start()
        idx_dma(1, 1).start()
        idx_dma(0, 0).wait()
        rows_dma(0).start()

        def pair(m, carry):
            for p in (0, 1):
                n = 2 * m + p
                if p == 0:
                    idx_dma(n + 1, 1).wait()
                    rows_dma(1).start()
                else:
                    @pl.when(m < NT // 2 - 1)
                    def _():
                        idx_dma(n + 1, 0).wait()
                        rows_dma(0).start()
                rows_dma(p).wait()
                @pl.when(m < NT // 2 - 1)
                def _():
                    idx_dma(n + 2, p).start()
                @pl.when(m >= 1)
                def _():
                    out_dma(n - 2, p).wait()
                compute(p)
                out_dma(n, p).start()
            return carry

        lax.fori_loop(0, NT // 2, pair, 0)
        out_dma(NT - 2, 0).wait()
        out_dma(NT - 1, 1).wait()

    return sc_fn(x, nbr_flat)


def _mm_body(x_ref, n_ref, w_ref, b_ref, o_ref):
    s = x_ref[...] + n_ref[...]
    acc = lax.dot_general(
        s, w_ref[...], (((1,), (1,)), ((), ())),
        preferred_element_type=jnp.float32,
    )
    o_ref[...] = acc + (K + 1.0) * b_ref[...]


def _tc_matmul(x, n, w, b):
    BM = 2000
    return pl.pallas_call(
        _mm_body,
        grid=(E // BM,),
        in_specs=[
            pl.BlockSpec((BM, D), lambda i: (i, 0)),
            pl.BlockSpec((BM, D), lambda i: (i, 0)),
            pl.BlockSpec((D, D), lambda i: (0, 0)),
            pl.BlockSpec((1, D), lambda i: (0, 0)),
        ],
        out_specs=pl.BlockSpec((BM, D), lambda i: (i, 0)),
        out_shape=jax.ShapeDtypeStruct((E, D), jnp.float32),
    )(x, n, w, b.reshape(1, D))


def kernel(edge_feats, neighbors, W, b):
    nbr_flat = neighbors.astype(jnp.int32).reshape(E * K)
    n_sum = _sc_gather_sum(edge_feats, nbr_flat)
    return _tc_matmul(edge_feats, n_sum, W, b)
